# Initial kernel scaffold; baseline (speedup 1.0000x reference)
import jax
import jax.numpy as jnp
from jax.experimental import pallas as pl


def kernel(queries, keys, values, obs, W1, b1, W2, b2):
    raise NotImplementedError("write your pallas kernel here")


# speedup vs baseline: 2.9703x; 2.9703x over previous
import functools

import jax
import jax.numpy as jnp
from jax import lax
from jax.experimental import pallas as pl
from jax.experimental.pallas import tpu as pltpu
from jax.experimental.pallas import tpu_sc as plsc

_TK = 2048


def _topk_body(K, TK, q_ref, kt_ref, idx_out, vmax_ref):
    i = pl.program_id(0)

    @pl.when(i == 0)
    def _init():
        vmax_ref[:] = jnp.full_like(vmax_ref, -jnp.inf)
        idx_out[:] = jnp.zeros_like(idx_out)

    s = jnp.dot(q_ref[:], kt_ref[:], preferred_element_type=jnp.float32)
    cols = i * TK + lax.broadcasted_iota(jnp.int32, s.shape, 1)
    s = jnp.where(cols < K, s, -jnp.inf)
    tmax = jnp.max(s, axis=1, keepdims=True)
    big = jnp.iinfo(jnp.int32).max
    targ = jnp.min(jnp.where(s == tmax, cols, big), axis=1, keepdims=True)
    better = tmax > vmax_ref[:]
    idx_out[:] = jnp.where(better, targ, idx_out[:])
    vmax_ref[:] = jnp.where(better, tmax, vmax_ref[:])


def _fused_top1(queries, keys):
    B, D = queries.shape
    K = keys.shape[0]
    keys_t = keys.T
    nsteps = (K + _TK - 1) // _TK
    idx2d = pl.pallas_call(
        functools.partial(_topk_body, K, _TK),
        grid=(nsteps,),
        in_specs=[
            pl.BlockSpec((B, D), lambda i: (0, 0)),
            pl.BlockSpec((D, _TK), lambda i: (0, i)),
        ],
        out_specs=pl.BlockSpec((B, 1), lambda i: (0, 0)),
        out_shape=jax.ShapeDtypeStruct((B, 1), jnp.int32),
        scratch_shapes=[pltpu.VMEM((B, 1), jnp.float32)],
    )(queries, keys_t)
    return idx2d.reshape(B)


def _sc_gather(table, idx):
    V, D = table.shape
    B = idx.shape[0]
    info = plsc.get_sparse_core_info()
    NC, NS = info.num_cores, info.num_subcores
    NW = NC * NS
    b_per_w = B // NW
    mesh = plsc.VectorSubcoreMesh(core_axis_name="c", subcore_axis_name="s")

    @functools.partial(
        pl.kernel,
        mesh=mesh,
        out_type=jax.ShapeDtypeStruct((B, D), jnp.float32),
        scratch_types=[
            pltpu.VMEM((b_per_w,), jnp.int32),
            pltpu.VMEM((b_per_w, D), jnp.float32),
            pltpu.SemaphoreType.DMA,
        ],
        compiler_params=pltpu.CompilerParams(use_tc_tiling_on_sc=False),
    )
    def gather_kernel(table_hbm, idx_hbm, out_hbm, idx_v, rows_v, sem):
        wid = lax.axis_index("s") * NC + lax.axis_index("c")
        base = wid * b_per_w
        pltpu.sync_copy(idx_hbm.at[pl.ds(base, b_per_w)], idx_v)
        pltpu.async_copy(table_hbm.at[idx_v], rows_v, sem).wait()
        pltpu.sync_copy(rows_v, out_hbm.at[pl.ds(base, b_per_w)])

    return gather_kernel(table, idx)


def _mlp_body(obs_ref, ctx_ref, w1o_ref, w1c_ref, b1_ref, w2_ref, b2_ref, q_out):
    h = jnp.dot(obs_ref[:], w1o_ref[:], preferred_element_type=jnp.float32)
    h = h + jnp.dot(ctx_ref[:], w1c_ref[:], preferred_element_type=jnp.float32)
    h = jnp.maximum(h + b1_ref[:], 0.0)
    q_out[:] = jnp.dot(h, w2_ref[:], preferred_element_type=jnp.float32) + b2_ref[:]


def _mlp(obs, ctx, W1, b1, W2, b2):
    B, OBS = obs.shape
    D = ctx.shape[1]
    H = W1.shape[1]
    A = W2.shape[1]
    return pl.pallas_call(
        _mlp_body,
        out_shape=jax.ShapeDtypeStruct((B, A), jnp.float32),
    )(obs, ctx, W1[:OBS], W1[OBS:], b1.reshape(1, H), W2, b2.reshape(1, A))


def kernel(queries, keys, values, obs, W1, b1, W2, b2):
    top_idx = _fused_top1(queries, keys)
    ctx = _sc_gather(values, top_idx)
    return _mlp(obs, ctx, W1, b1, W2, b2)
